# baseline (speedup 1.0000x reference)
import functools

import jax
import jax.numpy as jnp
from jax import lax
from jax.experimental import pallas as pl
from jax.experimental.pallas import tpu as pltpu
from jax.experimental.pallas import tpu_sc as plsc

B = 4096
H = 64
K = 100
NW = 32
BPW = B // NW


def _sc_gather(users, pos_items, item_idx, user_table, item_table):
    mesh = plsc.VectorSubcoreMesh(core_axis_name="c", subcore_axis_name="s")
    nc = mesh.num_cores

    @functools.partial(
        pl.kernel,
        out_type=[
            jax.ShapeDtypeStruct((B, H), jnp.float32),
            jax.ShapeDtypeStruct((B, H), jnp.float32),
            jax.ShapeDtypeStruct((B * K, H), jnp.float32),
        ],
        mesh=mesh,
        compiler_params=pltpu.CompilerParams(use_tc_tiling_on_sc=False),
        scratch_types=[
            pltpu.VMEM((BPW,), jnp.int32),
            pltpu.VMEM((BPW,), jnp.int32),
            pltpu.VMEM((BPW, K), jnp.int32),
            pltpu.VMEM((BPW, H), jnp.float32),
            pltpu.VMEM((BPW, H), jnp.float32),
            pltpu.VMEM((K, H), jnp.float32),
            pltpu.VMEM((K, H), jnp.float32),
            pltpu.SemaphoreType.DMA,
            pltpu.SemaphoreType.DMA,
        ],
    )
    def k(users_hbm, pos_hbm, iidx_hbm, utab_hbm, itab_hbm,
          uemb_out, pemb_out, rows_out,
          uidx_v, pidx_v, iidx_v, urows_v, prows_v, bufa, bufb, sema, semb):
        wid = lax.axis_index("s") * nc + lax.axis_index("c")
        base = wid * BPW

        pltpu.sync_copy(users_hbm.at[pl.ds(base, BPW)], uidx_v)
        pltpu.sync_copy(pos_hbm.at[pl.ds(base, BPW)], pidx_v)
        pltpu.sync_copy(iidx_hbm.at[pl.ds(base, BPW)], iidx_v)

        pltpu.make_async_copy(utab_hbm.at[uidx_v], urows_v, sema).start()
        pltpu.make_async_copy(itab_hbm.at[pidx_v], prows_v, semb).start()
        pltpu.make_async_copy(utab_hbm.at[uidx_v], urows_v, sema).wait()
        pltpu.sync_copy(urows_v, uemb_out.at[pl.ds(base, BPW)])
        pltpu.make_async_copy(itab_hbm.at[pidx_v], prows_v, semb).wait()
        pltpu.sync_copy(prows_v, pemb_out.at[pl.ds(base, BPW)])

        def gather(b_slot, buf, sem):
            pltpu.make_async_copy(itab_hbm.at[iidx_v.at[b_slot]], buf, sem).start()

        def wait_and_flush(b_slot, buf, sem):
            pltpu.make_async_copy(itab_hbm.at[iidx_v.at[b_slot]], buf, sem).wait()
            pltpu.sync_copy(buf, rows_out.at[pl.ds((base + b_slot) * K, K)])

        gather(0, bufa, sema)

        def body(g, _):
            i0 = 2 * g
            gather(i0 + 1, bufb, semb)
            wait_and_flush(i0, bufa, sema)

            @pl.when(i0 + 2 < BPW)
            def _():
                gather(i0 + 2, bufa, sema)

            wait_and_flush(i0 + 1, bufb, semb)
            return 0

        lax.fori_loop(0, BPW // 2, body, 0)

    return k(users, pos_items, item_idx, user_table, item_table)


def _tc_rating_body(u_ref, it_ref, out_ref):
    u = u_ref[...]
    it = it_ref[...]
    out_ref[...] = jnp.sum(it * u[:, None, :], axis=-1)


def _tc_rating(user_emb, all_items):
    bb = 256
    return pl.pallas_call(
        _tc_rating_body,
        grid=(B // bb,),
        in_specs=[
            pl.BlockSpec((bb, H), lambda i: (i, 0)),
            pl.BlockSpec((bb, K, H), lambda i: (i, 0, 0)),
        ],
        out_specs=pl.BlockSpec((bb, K), lambda i: (i, 0)),
        out_shape=jax.ShapeDtypeStruct((B, K), jnp.float32),
    )(user_emb, all_items)


def kernel(users, pos_items, neg_items, user_table, item_table):
    users = users.astype(jnp.int32)
    pos_items = pos_items.astype(jnp.int32)
    item_idx = jnp.concatenate(
        [pos_items[:, None], neg_items.astype(jnp.int32)], axis=1)

    user_emb, pos_emb, all_rows = _sc_gather(
        users, pos_items, item_idx, user_table, item_table)
    all_items_emb = all_rows.reshape(B, K, H)
    rating = _tc_rating(user_emb, all_items_emb)
    return (user_emb, pos_emb, all_items_emb, rating)


# speedup vs baseline: 1.5195x; 1.5195x over previous
import functools

import jax
import jax.numpy as jnp
from jax import lax
from jax.experimental import pallas as pl
from jax.experimental.pallas import tpu as pltpu
from jax.experimental.pallas import tpu_sc as plsc

B = 4096
H = 64
K = 100
NW = 32
BPW = B // NW


def _sc_gather_pairs(pair_t, upair, t2i, t2u):
    mesh = plsc.VectorSubcoreMesh(core_axis_name="c", subcore_axis_name="s")
    nc = mesh.num_cores

    @functools.partial(
        pl.kernel,
        out_type=[
            jax.ShapeDtypeStruct((K, B, 128), jnp.float32),
            jax.ShapeDtypeStruct((B, 128), jnp.float32),
        ],
        mesh=mesh,
        scratch_types=[
            pltpu.VMEM((K, BPW), jnp.int32),
            pltpu.VMEM((BPW,), jnp.int32),
            pltpu.VMEM((BPW, 128), jnp.float32),
            pltpu.VMEM((BPW, 128), jnp.float32),
            pltpu.SemaphoreType.DMA,
            pltpu.SemaphoreType.DMA,
        ],
    )
    def k(pair_hbm, upair_hbm, t2i_hbm, t2u_hbm, pairs_out, upairs_out,
          pv, upv, ra, rb, sema, semb):
        wid = lax.axis_index("s") * nc + lax.axis_index("c")
        b0 = wid * BPW

        pltpu.sync_copy(pair_hbm.at[:, pl.ds(b0, BPW)], pv)
        pltpu.sync_copy(upair_hbm.at[pl.ds(b0, BPW)], upv)

        pltpu.make_async_copy(t2u_hbm.at[upv], ra, sema).start()
        pltpu.make_async_copy(t2u_hbm.at[upv], ra, sema).wait()
        pltpu.sync_copy(ra, upairs_out.at[pl.ds(b0, BPW)])

        def gather(k_slot, buf, sem):
            pltpu.make_async_copy(t2i_hbm.at[pv.at[k_slot]], buf, sem).start()

        def flush(k_slot, buf, sem):
            pltpu.make_async_copy(t2i_hbm.at[pv.at[k_slot]], buf, sem).wait()
            pltpu.sync_copy(buf, pairs_out.at[k_slot, pl.ds(b0, BPW), :])

        gather(0, ra, sema)

        def body(j, _):
            k0 = 2 * j
            gather(k0 + 1, rb, semb)
            flush(k0, ra, sema)

            @pl.when(k0 + 2 < K)
            def _():
                gather(k0 + 2, ra, sema)

            flush(k0 + 1, rb, semb)
            return 0

        lax.fori_loop(0, K // 2, body, 0)

    return k(pair_t, upair, t2i, t2u)


def _tc_finish_body(idx_ref, users_ref, pairs_ref, upairs_ref,
                    uet_ref, allt_ref, rat_ref):
    up = upairs_ref[...]
    upt = up.T
    upar = (users_ref[...] & 1) == 1
    ue = jnp.where(upar, upt[H:], upt[:H])
    uet_ref[...] = ue

    x = pairs_ref[...]
    xt = jnp.transpose(x, (0, 2, 1))
    par = ((idx_ref[...] & 1) == 1)[:, None, :]
    sel = jnp.where(par, xt[:, H:, :], xt[:, :H, :])
    allt_ref[...] = sel
    rat_ref[...] = jnp.sum(sel * ue[None], axis=1)


def _tc_finish(idx_t, users2d, pairs, upairs):
    bb = 128
    return pl.pallas_call(
        _tc_finish_body,
        grid=(B // bb,),
        in_specs=[
            pl.BlockSpec((K, bb), lambda i: (0, i)),
            pl.BlockSpec((1, bb), lambda i: (0, i)),
            pl.BlockSpec((K, bb, 128), lambda i: (0, i, 0)),
            pl.BlockSpec((bb, 128), lambda i: (i, 0)),
        ],
        out_specs=[
            pl.BlockSpec((H, bb), lambda i: (0, i)),
            pl.BlockSpec((K, H, bb), lambda i: (0, 0, i)),
            pl.BlockSpec((K, bb), lambda i: (0, i)),
        ],
        out_shape=[
            jax.ShapeDtypeStruct((H, B), jnp.float32),
            jax.ShapeDtypeStruct((K, H, B), jnp.float32),
            jax.ShapeDtypeStruct((K, B), jnp.float32),
        ],
    )(idx_t, users2d, pairs, upairs)


def kernel(users, pos_items, neg_items, user_table, item_table):
    users = users.astype(jnp.int32)
    pos_items = pos_items.astype(jnp.int32)
    item_idx = jnp.concatenate(
        [pos_items[:, None], neg_items.astype(jnp.int32)], axis=1)
    idx_t = item_idx.T
    pair_t = jax.lax.shift_right_logical(idx_t, 1)
    upair = jax.lax.shift_right_logical(users, 1)

    t2i = item_table.reshape(500000, 128)
    t2u = user_table.reshape(500000, 128)

    pairs, upairs = _sc_gather_pairs(pair_t, upair, t2i, t2u)
    ue_t, all_t, rating_t = _tc_finish(idx_t, users[None, :], pairs, upairs)
    pe_t = all_t[0]

    return (ue_t.T, pe_t.T, all_t.transpose(2, 0, 1), rating_t.T)


# speedup vs baseline: 2.3546x; 1.5496x over previous
import functools

import jax
import jax.numpy as jnp
from jax import lax
from jax.experimental import pallas as pl
from jax.experimental.pallas import tpu as pltpu
from jax.experimental.pallas import tpu_sc as plsc

B = 4096
H = 64
K = 100
NW = 32
BPW = B // NW
NT = 1000000
HALF = 1 << 19
BL = 2048


def _tc_build_pairs_body(x0_ref, x1_ref, out_ref):
    out_ref[:, :H] = x0_ref[...].T
    out_ref[:, H:] = x1_ref[...].T


def _tc_build_pairs(table_t):
    return pl.pallas_call(
        _tc_build_pairs_body,
        grid=(HALF // BL,),
        in_specs=[
            pl.BlockSpec((H, BL), lambda j: (0, j)),
            pl.BlockSpec(
                (H, BL),
                lambda j: (0, jnp.minimum(j + HALF // BL,
                                          (NT + BL - 1) // BL - 1))),
        ],
        out_specs=pl.BlockSpec((BL, 2 * H), lambda j: (j, 0)),
        out_shape=jax.ShapeDtypeStruct((HALF, 2 * H), jnp.float32),
    )(table_t, table_t)


def _sc_mesh():
    return plsc.VectorSubcoreMesh(core_axis_name="c", subcore_axis_name="s")


def _sc_gather_items(pair_t, t2i):
    mesh = _sc_mesh()
    nc = mesh.num_cores

    @functools.partial(
        pl.kernel,
        out_type=jax.ShapeDtypeStruct((K, B, 128), jnp.float32),
        mesh=mesh,
        scratch_types=[
            pltpu.VMEM((K, BPW), jnp.int32),
            pltpu.VMEM((BPW, 128), jnp.float32),
            pltpu.VMEM((BPW, 128), jnp.float32),
            pltpu.SemaphoreType.DMA,
            pltpu.SemaphoreType.DMA,
        ],
    )
    def k(pair_hbm, t2i_hbm, pairs_out, pv, ra, rb, sema, semb):
        wid = lax.axis_index("s") * nc + lax.axis_index("c")
        b0 = wid * BPW

        pltpu.sync_copy(pair_hbm.at[:, pl.ds(b0, BPW)], pv)

        def gather(k_slot, buf, sem):
            pltpu.make_async_copy(t2i_hbm.at[pv.at[k_slot]], buf, sem).start()

        def flush(k_slot, buf, sem):
            pltpu.make_async_copy(t2i_hbm.at[pv.at[k_slot]], buf, sem).wait()
            pltpu.sync_copy(buf, pairs_out.at[k_slot, pl.ds(b0, BPW), :])

        gather(0, ra, sema)

        def body(j, _):
            k0 = 2 * j
            gather(k0 + 1, rb, semb)
            flush(k0, ra, sema)

            @pl.when(k0 + 2 < K)
            def _():
                gather(k0 + 2, ra, sema)

            flush(k0 + 1, rb, semb)
            return 0

        lax.fori_loop(0, K // 2, body, 0)

    return k(pair_t, t2i)


def _sc_gather_users(upair, t2u):
    mesh = _sc_mesh()
    nc = mesh.num_cores

    @functools.partial(
        pl.kernel,
        out_type=jax.ShapeDtypeStruct((B, 128), jnp.float32),
        mesh=mesh,
        scratch_types=[
            pltpu.VMEM((BPW,), jnp.int32),
            pltpu.VMEM((BPW, 128), jnp.float32),
            pltpu.SemaphoreType.DMA,
        ],
    )
    def k(upair_hbm, t2u_hbm, upairs_out, upv, ra, sema):
        wid = lax.axis_index("s") * nc + lax.axis_index("c")
        b0 = wid * BPW
        pltpu.sync_copy(upair_hbm.at[pl.ds(b0, BPW)], upv)
        pltpu.make_async_copy(t2u_hbm.at[upv], ra, sema).start()
        pltpu.make_async_copy(t2u_hbm.at[upv], ra, sema).wait()
        pltpu.sync_copy(ra, upairs_out.at[pl.ds(b0, BPW)])

    return k(upair, t2u)


def _tc_finish_body(idx_ref, users_ref, pairs_ref, upairs_ref,
                    uet_ref, allt_ref, rat_ref):
    up = upairs_ref[...]
    upt = up.T
    upar = users_ref[...] >= HALF
    ue = jnp.where(upar, upt[H:], upt[:H])
    uet_ref[...] = ue

    x = pairs_ref[...]
    xt = jnp.transpose(x, (0, 2, 1))
    par = (idx_ref[...] >= HALF)[:, None, :]
    sel = jnp.where(par, xt[:, H:, :], xt[:, :H, :])
    allt_ref[...] = sel
    rat_ref[...] = jnp.sum(sel * ue[None], axis=1)


def _tc_finish(idx_t, users2d, pairs, upairs):
    bb = 128
    return pl.pallas_call(
        _tc_finish_body,
        grid=(B // bb,),
        in_specs=[
            pl.BlockSpec((K, bb), lambda i: (0, i)),
            pl.BlockSpec((1, bb), lambda i: (0, i)),
            pl.BlockSpec((K, bb, 128), lambda i: (0, i, 0)),
            pl.BlockSpec((bb, 128), lambda i: (i, 0)),
        ],
        out_specs=[
            pl.BlockSpec((H, bb), lambda i: (0, i)),
            pl.BlockSpec((K, H, bb), lambda i: (0, 0, i)),
            pl.BlockSpec((K, bb), lambda i: (0, i)),
        ],
        out_shape=[
            jax.ShapeDtypeStruct((H, B), jnp.float32),
            jax.ShapeDtypeStruct((K, H, B), jnp.float32),
            jax.ShapeDtypeStruct((K, B), jnp.float32),
        ],
    )(idx_t, users2d, pairs, upairs)


def kernel(users, pos_items, neg_items, user_table, item_table):
    users = users.astype(jnp.int32)
    pos_items = pos_items.astype(jnp.int32)
    item_idx = jnp.concatenate(
        [pos_items[:, None], neg_items.astype(jnp.int32)], axis=1)
    idx_t = item_idx.T
    pair_t = idx_t & (HALF - 1)
    upair = users & (HALF - 1)

    t2i = _tc_build_pairs(item_table.T)
    t2u = _tc_build_pairs(user_table.T)

    pairs = _sc_gather_items(pair_t, t2i)
    upairs = _sc_gather_users(upair, t2u)
    ue_t, all_t, rating_t = _tc_finish(idx_t, users[None, :], pairs, upairs)
    pe_t = all_t[0]

    return (ue_t.T, pe_t.T, all_t.transpose(2, 0, 1), rating_t.T)


# speedup vs baseline: 2.7617x; 1.1729x over previous
import functools

import jax
import jax.numpy as jnp
from jax import lax
from jax.experimental import pallas as pl
from jax.experimental.pallas import tpu as pltpu
from jax.experimental.pallas import tpu_sc as plsc

B = 4096
H = 64
K = 100
NW = 32
BPW = B // NW
NT = 1000000
HALF = 1 << 19
BL = 4096


def _tc_build_pairs_body(x0_ref, x1_ref, out_ref):
    out_ref[...] = jnp.concatenate([x0_ref[...].T, x1_ref[...].T], axis=1)


def _tc_build_pairs(table_t):
    return pl.pallas_call(
        _tc_build_pairs_body,
        grid=(HALF // BL,),
        in_specs=[
            pl.BlockSpec((H, BL), lambda j: (0, j)),
            pl.BlockSpec(
                (H, BL),
                lambda j: (0, jnp.minimum(j + HALF // BL,
                                          (NT + BL - 1) // BL - 1))),
        ],
        out_specs=pl.BlockSpec((BL, 2 * H), lambda j: (j, 0)),
        out_shape=jax.ShapeDtypeStruct((HALF, 2 * H), jnp.float32),
    )(table_t, table_t)


def _sc_mesh():
    return plsc.VectorSubcoreMesh(core_axis_name="c", subcore_axis_name="s")


def _sc_gather_items(pair_t, t2i):
    mesh = _sc_mesh()
    nc = mesh.num_cores

    @functools.partial(
        pl.kernel,
        out_type=jax.ShapeDtypeStruct((K, B, 128), jnp.float32),
        mesh=mesh,
        scratch_types=[
            pltpu.VMEM((K, BPW), jnp.int32),
            pltpu.VMEM((BPW, 128), jnp.float32),
            pltpu.VMEM((BPW, 128), jnp.float32),
            pltpu.SemaphoreType.DMA,
            pltpu.SemaphoreType.DMA,
        ],
    )
    def k(pair_hbm, t2i_hbm, pairs_out, pv, ra, rb, sema, semb):
        wid = lax.axis_index("s") * nc + lax.axis_index("c")
        b0 = wid * BPW

        pltpu.sync_copy(pair_hbm.at[:, pl.ds(b0, BPW)], pv)

        def gather(k_slot, buf, sem):
            pltpu.make_async_copy(t2i_hbm.at[pv.at[k_slot]], buf, sem).start()

        def flush(k_slot, buf, sem):
            pltpu.make_async_copy(t2i_hbm.at[pv.at[k_slot]], buf, sem).wait()
            pltpu.sync_copy(buf, pairs_out.at[k_slot, pl.ds(b0, BPW), :])

        gather(0, ra, sema)

        def body(j, _):
            k0 = 2 * j
            gather(k0 + 1, rb, semb)
            flush(k0, ra, sema)

            @pl.when(k0 + 2 < K)
            def _():
                gather(k0 + 2, ra, sema)

            flush(k0 + 1, rb, semb)
            return 0

        lax.fori_loop(0, K // 2, body, 0)

    return k(pair_t, t2i)


def _sc_gather_users(upair, t2u):
    mesh = _sc_mesh()
    nc = mesh.num_cores

    @functools.partial(
        pl.kernel,
        out_type=jax.ShapeDtypeStruct((B, 128), jnp.float32),
        mesh=mesh,
        scratch_types=[
            pltpu.VMEM((BPW,), jnp.int32),
            pltpu.VMEM((BPW, 128), jnp.float32),
            pltpu.SemaphoreType.DMA,
        ],
    )
    def k(upair_hbm, t2u_hbm, upairs_out, upv, ra, sema):
        wid = lax.axis_index("s") * nc + lax.axis_index("c")
        b0 = wid * BPW
        pltpu.sync_copy(upair_hbm.at[pl.ds(b0, BPW)], upv)
        pltpu.make_async_copy(t2u_hbm.at[upv], ra, sema).start()
        pltpu.make_async_copy(t2u_hbm.at[upv], ra, sema).wait()
        pltpu.sync_copy(ra, upairs_out.at[pl.ds(b0, BPW)])

    return k(upair, t2u)


def _tc_finish_body(idx_ref, users_ref, pairs_ref, upairs_ref,
                    uet_ref, allt_ref, rat_ref):
    up = upairs_ref[...]
    upt = up.T
    upar = users_ref[...] >= HALF
    ue = jnp.where(upar, upt[H:], upt[:H])
    uet_ref[...] = ue

    x = pairs_ref[...]
    xt = jnp.transpose(x, (0, 2, 1))
    par = (idx_ref[...] >= HALF)[:, None, :]
    sel = jnp.where(par, xt[:, H:, :], xt[:, :H, :])
    allt_ref[...] = sel
    rat_ref[...] = jnp.sum(sel * ue[None], axis=1)


def _tc_finish(idx_t, users2d, pairs, upairs):
    bb = 128
    return pl.pallas_call(
        _tc_finish_body,
        grid=(B // bb,),
        in_specs=[
            pl.BlockSpec((K, bb), lambda i: (0, i)),
            pl.BlockSpec((1, bb), lambda i: (0, i)),
            pl.BlockSpec((K, bb, 128), lambda i: (0, i, 0)),
            pl.BlockSpec((bb, 128), lambda i: (i, 0)),
        ],
        out_specs=[
            pl.BlockSpec((H, bb), lambda i: (0, i)),
            pl.BlockSpec((K, H, bb), lambda i: (0, 0, i)),
            pl.BlockSpec((K, bb), lambda i: (0, i)),
        ],
        out_shape=[
            jax.ShapeDtypeStruct((H, B), jnp.float32),
            jax.ShapeDtypeStruct((K, H, B), jnp.float32),
            jax.ShapeDtypeStruct((K, B), jnp.float32),
        ],
    )(idx_t, users2d, pairs, upairs)


def kernel(users, pos_items, neg_items, user_table, item_table):
    users = users.astype(jnp.int32)
    pos_items = pos_items.astype(jnp.int32)
    item_idx = jnp.concatenate(
        [pos_items[:, None], neg_items.astype(jnp.int32)], axis=1)
    idx_t = item_idx.T
    pair_t = idx_t & (HALF - 1)
    upair = users & (HALF - 1)

    t2i = _tc_build_pairs(item_table.T)
    t2u = _tc_build_pairs(user_table.T)

    pairs = _sc_gather_items(pair_t, t2i)
    upairs = _sc_gather_users(upair, t2u)
    ue_t, all_t, rating_t = _tc_finish(idx_t, users[None, :], pairs, upairs)
    pe_t = all_t[0]

    return (ue_t.T, pe_t.T, all_t.transpose(2, 0, 1), rating_t.T)
